# baseline (speedup 1.0000x reference)
import functools

import numpy as np

import jax
import jax.numpy as jnp
from jax import lax
from jax.experimental import pallas as pl
from jax.experimental.pallas import tpu as pltpu
from jax.experimental.pallas import tpu_sc as plsc

_VOCAB = 38462 * 26
_D = 16
_B = 16384
_F = 26
_N = _B * _F

_INFO = plsc.get_sparse_core_info()
_NC = _INFO.num_cores
_NS = _INFO.num_subcores
_NW = _NC * _NS
_BB = 128
_CH = _F * _BB
_NT = _B // _BB
_TPW = _NT // _NW
_SUB = 128
_NSUB = _CH // _SUB
_NG = _SUB // 16

@functools.partial(
    pl.kernel,
    mesh=plsc.VectorSubcoreMesh(core_axis_name="c", subcore_axis_name="s"),
    out_type=jax.ShapeDtypeStruct((_F * _D, _B), jnp.float32),
    scratch_types=[
        pltpu.VMEM((_CH,), jnp.int32),
        pltpu.VMEM((2, _SUB, _D), jnp.float32),
        pltpu.VMEM((_F * _D, _BB), jnp.float32),
        pltpu.SemaphoreType.DMA,
        pltpu.SemaphoreType.DMA,
        pltpu.SemaphoreType.DMA,
    ],
    compiler_params=pltpu.CompilerParams(
        use_tc_tiling_on_sc=False, needs_layout_passes=False),
)
def _sc_embed(xf_hbm, tab_hbm, outT_hbm, idx_v, rows_v, sbuf, gsem0, gsem1, wsem):
    wid = lax.axis_index("s") * _NC + lax.axis_index("c")
    iota16 = lax.iota(jnp.int32, 16)
    gsems = (gsem0, gsem1)
    rowsel = [iota16 + (g * 16) for g in range(_NG)]

    def fire(s, p):
        return pltpu.async_copy(
            tab_hbm.at[idx_v.at[pl.ds(s * _SUB, _SUB)]], rows_v.at[p], gsems[p])

    def drain(p):
        pltpu.make_async_copy(
            tab_hbm.at[idx_v.at[pl.ds(0, _SUB)]], rows_v.at[p], gsems[p]).wait()

    def redistribute(s, p):
        psel = jnp.full((16,), p, jnp.int32)
        for g in range(_NG):
            qvec = iota16 + (s * _SUB + g * 16)
            db = qvec // _F
            f16 = (qvec - db * _F) * _D
            for e in range(_D):
                vals = plsc.load_gather(
                    rows_v, [psel, rowsel[g], jnp.full((16,), e, jnp.int32)])
                plsc.store_scatter(sbuf, [f16 + e, db], vals)

    def task(t_local, carry):
        t = wid * _TPW + t_local

        pltpu.sync_copy(xf_hbm.at[pl.ds(t * _CH, _CH)], idx_v)
        fire(0, 0)

        @pl.when(t_local > 0)
        def _():
            pltpu.make_async_copy(
                sbuf, outT_hbm.at[:, pl.ds(0, _BB)], wsem).wait()

        def pair(i, c):
            fire(2 * i + 1, 1)
            drain(0)
            redistribute(2 * i, 0)

            @pl.when(i < _NSUB // 2 - 1)
            def _():
                fire(2 * i + 2, 0)

            drain(1)
            redistribute(2 * i + 1, 1)
            return c

        lax.fori_loop(0, _NSUB // 2, pair, 0)

        pltpu.async_copy(sbuf, outT_hbm.at[:, pl.ds(t * _BB, _BB)], wsem)
        return carry

    lax.fori_loop(0, _TPW, task, 0)
    pltpu.make_async_copy(sbuf, outT_hbm.at[:, pl.ds(0, _BB)], wsem).wait()


def kernel(x, table):
    xf = x.astype(jnp.int32).reshape(_N)
    outT = _sc_embed(xf, table)
    return outT.reshape(_F, _D, _B).transpose(2, 0, 1)


# speedup vs baseline: 1.0003x; 1.0003x over previous
import functools

import numpy as np

import jax
import jax.numpy as jnp
from jax import lax
from jax.experimental import pallas as pl
from jax.experimental.pallas import tpu as pltpu
from jax.experimental.pallas import tpu_sc as plsc

_VOCAB = 38462 * 26
_D = 16
_B = 16384
_F = 26
_N = _B * _F

_INFO = plsc.get_sparse_core_info()
_NC = _INFO.num_cores
_NS = _INFO.num_subcores
_NW = _NC * _NS
_BB = 128
_CH = _F * _BB
_NT = _B // _BB
_TPW = _NT // _NW
_SUB = 128
_NSUB = _CH // _SUB
_NG = _SUB // 16

@functools.partial(
    pl.kernel,
    mesh=plsc.VectorSubcoreMesh(core_axis_name="c", subcore_axis_name="s"),
    out_type=jax.ShapeDtypeStruct((_F * _D, _B), jnp.float32),
    scratch_types=[
        pltpu.VMEM((_NSUB, _SUB), jnp.int32),
        pltpu.VMEM((2, _SUB, _D), jnp.float32),
        pltpu.VMEM((_F * _D, _BB), jnp.float32),
        pltpu.SemaphoreType.DMA,
        pltpu.SemaphoreType.DMA,
        pltpu.SemaphoreType.DMA,
    ],
    compiler_params=pltpu.CompilerParams(
        use_tc_tiling_on_sc=False, needs_layout_passes=False),
)
def _sc_embed(xf_hbm, tab_hbm, outT_hbm, idx_v, rows_v, sbuf, gsem0, gsem1, wsem):
    wid = lax.axis_index("s") * _NC + lax.axis_index("c")
    iota16 = lax.iota(jnp.int32, 16)
    gsems = (gsem0, gsem1)
    rowsel = [iota16 + (g * 16) for g in range(_NG)]

    def fire(s, p):
        return pltpu.async_copy(
            tab_hbm.at[idx_v.at[s]], rows_v.at[p], gsems[p])

    def drain(p):
        pltpu.make_async_copy(
            tab_hbm.at[idx_v.at[0]], rows_v.at[p], gsems[p]).wait()

    def redistribute(s, p):
        psel = jnp.full((16,), p, jnp.int32)
        for g in range(_NG):
            qvec = iota16 + (s * _SUB + g * 16)
            db = qvec // _F
            f16 = (qvec - db * _F) * _D
            for e in range(_D):
                vals = plsc.load_gather(
                    rows_v, [psel, rowsel[g], jnp.full((16,), e, jnp.int32)])
                plsc.store_scatter(sbuf, [f16 + e, db], vals)

    def task(t_local, carry):
        t = wid * _TPW + t_local

        pltpu.sync_copy(xf_hbm.at[pl.ds(t * _NSUB, _NSUB), :], idx_v)
        fire(0, 0)

        @pl.when(t_local > 0)
        def _():
            pltpu.make_async_copy(
                sbuf, outT_hbm.at[:, pl.ds(0, _BB)], wsem).wait()

        def pair(i, c):
            fire(2 * i + 1, 1)
            drain(0)
            redistribute(2 * i, 0)

            @pl.when(i < _NSUB // 2 - 1)
            def _():
                fire(2 * i + 2, 0)

            drain(1)
            redistribute(2 * i + 1, 1)
            return c

        lax.fori_loop(0, _NSUB // 2, pair, 0)

        pltpu.async_copy(sbuf, outT_hbm.at[:, pl.ds(t * _BB, _BB)], wsem)
        return carry

    lax.fori_loop(0, _TPW, task, 0)
    pltpu.make_async_copy(sbuf, outT_hbm.at[:, pl.ds(0, _BB)], wsem).wait()


def kernel(x, table):
    xf = x.astype(jnp.int32).reshape(_N // _SUB, _SUB)
    outT = _sc_embed(xf, table)
    return outT.reshape(_F, _D, _B).transpose(2, 0, 1)


# speedup vs baseline: 1.1220x; 1.1216x over previous
import functools

import jax
import jax.numpy as jnp
from jax import lax
from jax.experimental import pallas as pl
from jax.experimental.pallas import tpu as pltpu
from jax.experimental.pallas import tpu_sc as plsc

_VOCAB = 38462 * 26
_D = 16
_B = 16384
_F = 26

_INFO = plsc.get_sparse_core_info()
_NC = _INFO.num_cores
_NS = _INFO.num_subcores
_NW = _NC * _NS
_VPAD = 1000064
_NBLK = _VPAD // 128


@functools.partial(
    pl.kernel,
    mesh=plsc.VectorSubcoreMesh(core_axis_name="c", subcore_axis_name="s"),
    out_type=jax.ShapeDtypeStruct((_NBLK * 16, 128), jnp.float32),
    scratch_types=[
        pltpu.VMEM((2, 2, 8, 128), jnp.float32),
        pltpu.VMEM((2, 16, 128), jnp.float32),
        pltpu.SemaphoreType.DMA,
        pltpu.SemaphoreType.DMA,
        pltpu.SemaphoreType.DMA,
        pltpu.SemaphoreType.DMA,
    ],
    compiler_params=pltpu.CompilerParams(
        use_tc_tiling_on_sc=True, needs_layout_passes=False),
)
def _sc_retile(tT_hbm, flat8_hbm, vin, vout, gsemA, gsemB, wsemA, wsemB):
    wid = lax.axis_index("s") * _NC + lax.axis_index("c")
    ee = lax.iota(jnp.int32, 16)
    gsems = (gsemA, gsemB)
    wsems = (wsemA, wsemB)
    niter = (_NBLK + _NW - 1) // _NW

    def fire(i, p):
        j = wid + i * _NW

        @pl.when(j < _NBLK)
        def _():
            pltpu.async_copy(
                tT_hbm.at[pl.ds(0, 8), pl.ds(j * 128, 128)], vin.at[p, 0], gsems[p])
            pltpu.async_copy(
                tT_hbm.at[pl.ds(8, 8), pl.ds(j * 128, 128)], vin.at[p, 1], gsems[p])

    fire(0, 0)

    def step(i, carry):
        p = lax.rem(i, 2)

        @pl.when(lax.rem(i, 2) == 0)
        def _():
            _half(i, 0)

        @pl.when(lax.rem(i, 2) == 1)
        def _():
            _half(i, 1)

        return carry

    def _half(i, p):
        j = wid + i * _NW
        fire(i + 1, 1 - p)

        @pl.when(j < _NBLK)
        def _():
            pltpu.make_async_copy(
                tT_hbm.at[pl.ds(0, 8), pl.ds(0, 128)], vin.at[p, 0], gsems[p]).wait()
            pltpu.make_async_copy(
                tT_hbm.at[pl.ds(0, 8), pl.ds(0, 128)], vin.at[p, 1], gsems[p]).wait()
            @pl.when(i >= 2)
            def _():
                pltpu.make_async_copy(
                    vout.at[p], flat8_hbm.at[pl.ds(0, 16), :], wsems[p]).wait()

            for k in range(128):
                vals = plsc.load_gather(vin, [
                    jnp.full((16,), p, jnp.int32),
                    ee // 8, lax.rem(ee, 8),
                    jnp.full((16,), k, jnp.int32)])
                vout[p, k // 8, pl.ds((k % 8) * 16, 16)] = vals

            pltpu.async_copy(
                vout.at[p], flat8_hbm.at[pl.ds(16 * j, 16), :], wsems[p])

    lax.fori_loop(0, niter, step, 0)
    pltpu.make_async_copy(
        vout.at[0], flat8_hbm.at[pl.ds(0, 16), :], wsems[0]).wait()
    pltpu.make_async_copy(
        vout.at[1], flat8_hbm.at[pl.ds(0, 16), :], wsems[1]).wait()
_BC = 1024
_NT = _F * (_B // _BC)
_TPW = _NT // _NW
_SUB = 128
_NSUB = _BC // _SUB


@functools.partial(
    pl.kernel,
    mesh=plsc.VectorSubcoreMesh(core_axis_name="c", subcore_axis_name="s"),
    out_type=jax.ShapeDtypeStruct((_F, _D, _B), jnp.float32),
    scratch_types=[
        pltpu.VMEM((_BC,), jnp.int32),
        pltpu.VMEM((2, _SUB, _D), jnp.float32),
        pltpu.VMEM((_D, _BC), jnp.float32),
        pltpu.SemaphoreType.DMA,
        pltpu.SemaphoreType.DMA,
        pltpu.SemaphoreType.DMA,
    ],
    compiler_params=pltpu.CompilerParams(
        use_tc_tiling_on_sc=False, needs_layout_passes=False),
)
def _sc_embed(xT_hbm, tab_hbm, outT_hbm, idx_v, rows_v, tbuf, gsem0, gsem1, wsem):
    wid = lax.axis_index("s") * _NC + lax.axis_index("c")
    iota16 = lax.iota(jnp.int32, 16)
    gsems = (gsem0, gsem1)

    def task(t_local, carry):
        t = wid * _TPW + t_local
        f = t // (_B // _BC)
        c = lax.rem(t, _B // _BC)

        pltpu.sync_copy(xT_hbm.at[f, pl.ds(c * _BC, _BC)], idx_v)

        descs = [None] * _NSUB
        descs[0] = pltpu.async_copy(
            tab_hbm.at[idx_v.at[pl.ds(0, _SUB)]], rows_v.at[0], gsems[0])

        @pl.when(t_local > 0)
        def _():
            pltpu.make_async_copy(
                tbuf, outT_hbm.at[0, :, pl.ds(0, _BC)], wsem).wait()

        for s in range(_NSUB):
            p = s % 2
            if s + 1 < _NSUB:
                descs[s + 1] = pltpu.async_copy(
                    tab_hbm.at[idx_v.at[pl.ds((s + 1) * _SUB, _SUB)]],
                    rows_v.at[1 - p], gsems[(s + 1) % 2])
            descs[s].wait()
            for g in range(_SUB // 16):
                ridx = iota16 + (g * 16)
                for e in range(_D):
                    vals = plsc.load_gather(
                        rows_v,
                        [jnp.full((16,), p, jnp.int32), ridx,
                         jnp.full((16,), e, jnp.int32)])
                    tbuf[e, pl.ds(s * _SUB + g * 16, 16)] = vals

        pltpu.async_copy(tbuf, outT_hbm.at[f, :, pl.ds(c * _BC, _BC)], wsem)
        return carry

    lax.fori_loop(0, _TPW, task, 0)
    pltpu.make_async_copy(tbuf, outT_hbm.at[0, :, pl.ds(0, _BC)], wsem).wait()


def kernel(x, table):
    flat8 = _sc_retile(table.T)
    tab = flat8.reshape(_VPAD, _D)
    xT = x.T.astype(jnp.int32)
    outT = _sc_embed(xT, tab)
    return outT.transpose(2, 0, 1)


# speedup vs baseline: 1.7968x; 1.6014x over previous
import functools

import jax
import jax.numpy as jnp
from jax import lax
from jax.experimental import pallas as pl
from jax.experimental.pallas import tpu as pltpu
from jax.experimental.pallas import tpu_sc as plsc

_VOCAB = 38462 * 26
_D = 16
_B = 16384
_F = 26

_INFO = plsc.get_sparse_core_info()
_NC = _INFO.num_cores
_NS = _INFO.num_subcores
_NW = _NC * _NS
_VPAD = 1000064
_NBLK = _VPAD // 128


@functools.partial(
    pl.kernel,
    mesh=plsc.VectorSubcoreMesh(core_axis_name="c", subcore_axis_name="s"),
    out_type=jax.ShapeDtypeStruct((_NBLK * 16, 128), jnp.float32),
    scratch_types=[
        pltpu.VMEM((2, 2, 8, 128), jnp.float32),
        pltpu.VMEM((2, 16, 128), jnp.float32),
        pltpu.SemaphoreType.DMA,
        pltpu.SemaphoreType.DMA,
        pltpu.SemaphoreType.DMA,
        pltpu.SemaphoreType.DMA,
    ],
    compiler_params=pltpu.CompilerParams(
        use_tc_tiling_on_sc=True, needs_layout_passes=False),
)
def _sc_retile(tT_hbm, flat8_hbm, vin, vout, gsemA, gsemB, wsemA, wsemB):
    wid = lax.axis_index("s") * _NC + lax.axis_index("c")
    ee = lax.iota(jnp.int32, 16)
    gsems = (gsemA, gsemB)
    wsems = (wsemA, wsemB)
    niter = (_NBLK + _NW - 1) // _NW

    def fire(i, p):
        j = wid + i * _NW

        @pl.when(j < _NBLK)
        def _():
            pltpu.async_copy(
                tT_hbm.at[pl.ds(0, 8), pl.ds(j * 128, 128)], vin.at[p, 0], gsems[p])
            pltpu.async_copy(
                tT_hbm.at[pl.ds(8, 8), pl.ds(j * 128, 128)], vin.at[p, 1], gsems[p])

    fire(0, 0)

    def step(i, carry):
        p = lax.rem(i, 2)

        @pl.when(lax.rem(i, 2) == 0)
        def _():
            _half(i, 0)

        @pl.when(lax.rem(i, 2) == 1)
        def _():
            _half(i, 1)

        return carry

    def _half(i, p):
        j = wid + i * _NW
        fire(i + 1, 1 - p)

        @pl.when(j < _NBLK)
        def _():
            pltpu.make_async_copy(
                tT_hbm.at[pl.ds(0, 8), pl.ds(0, 128)], vin.at[p, 0], gsems[p]).wait()
            pltpu.make_async_copy(
                tT_hbm.at[pl.ds(0, 8), pl.ds(0, 128)], vin.at[p, 1], gsems[p]).wait()
            @pl.when(i >= 2)
            def _():
                pltpu.make_async_copy(
                    vout.at[p], flat8_hbm.at[pl.ds(0, 16), :], wsems[p]).wait()

            for k0 in range(0, 128, 8):
                vals = [plsc.load_gather(vin, [
                    jnp.full((16,), p, jnp.int32),
                    ee // 8, lax.rem(ee, 8),
                    jnp.full((16,), k0 + u, jnp.int32)]) for u in range(8)]
                for u in range(8):
                    k = k0 + u
                    vout[p, k // 8, pl.ds((k % 8) * 16, 16)] = vals[u]

            pltpu.async_copy(
                vout.at[p], flat8_hbm.at[pl.ds(16 * j, 16), :], wsems[p])

    lax.fori_loop(0, niter, step, 0)
    pltpu.make_async_copy(
        vout.at[0], flat8_hbm.at[pl.ds(0, 16), :], wsems[0]).wait()
    pltpu.make_async_copy(
        vout.at[1], flat8_hbm.at[pl.ds(0, 16), :], wsems[1]).wait()
_BC = 1024
_NT = _F * (_B // _BC)
_TPW = _NT // _NW
_SUB = 128
_NSUB = _BC // _SUB


@functools.partial(
    pl.kernel,
    mesh=plsc.VectorSubcoreMesh(core_axis_name="c", subcore_axis_name="s"),
    out_type=jax.ShapeDtypeStruct((_F, _D, _B), jnp.float32),
    scratch_types=[
        pltpu.VMEM((_BC,), jnp.int32),
        pltpu.VMEM((2, _SUB, _D), jnp.float32),
        pltpu.VMEM((_D, _BC), jnp.float32),
        pltpu.SemaphoreType.DMA,
        pltpu.SemaphoreType.DMA,
        pltpu.SemaphoreType.DMA,
    ],
    compiler_params=pltpu.CompilerParams(
        use_tc_tiling_on_sc=False, needs_layout_passes=False),
)
def _sc_embed(xT_hbm, tab_hbm, outT_hbm, idx_v, rows_v, tbuf, gsem0, gsem1, wsem):
    wid = lax.axis_index("s") * _NC + lax.axis_index("c")
    iota16 = lax.iota(jnp.int32, 16)
    gsems = (gsem0, gsem1)

    def task(t_local, carry):
        t = wid * _TPW + t_local
        f = t // (_B // _BC)
        c = lax.rem(t, _B // _BC)

        pltpu.sync_copy(xT_hbm.at[f, pl.ds(c * _BC, _BC)], idx_v)

        descs = [None] * _NSUB
        descs[0] = pltpu.async_copy(
            tab_hbm.at[idx_v.at[pl.ds(0, _SUB)]], rows_v.at[0], gsems[0])

        @pl.when(t_local > 0)
        def _():
            pltpu.make_async_copy(
                tbuf, outT_hbm.at[0, :, pl.ds(0, _BC)], wsem).wait()

        for s in range(_NSUB):
            p = s % 2
            if s + 1 < _NSUB:
                descs[s + 1] = pltpu.async_copy(
                    tab_hbm.at[idx_v.at[pl.ds((s + 1) * _SUB, _SUB)]],
                    rows_v.at[1 - p], gsems[(s + 1) % 2])
            descs[s].wait()
            for g in range(_SUB // 16):
                ridx = iota16 + (g * 16)
                for e0 in range(0, _D, 8):
                    vals = [plsc.load_gather(
                        rows_v,
                        [jnp.full((16,), p, jnp.int32), ridx,
                         jnp.full((16,), e0 + u, jnp.int32)]) for u in range(8)]
                    for u in range(8):
                        tbuf[e0 + u, pl.ds(s * _SUB + g * 16, 16)] = vals[u]

        pltpu.async_copy(tbuf, outT_hbm.at[f, :, pl.ds(c * _BC, _BC)], wsem)
        return carry

    lax.fori_loop(0, _TPW, task, 0)
    pltpu.make_async_copy(tbuf, outT_hbm.at[0, :, pl.ds(0, _BC)], wsem).wait()


def kernel(x, table):
    flat8 = _sc_retile(table.T)
    tab = flat8.reshape(_VPAD, _D)
    xT = x.T.astype(jnp.int32)
    outT = _sc_embed(xT, tab)
    return outT.transpose(2, 0, 1)
